# baseline (speedup 1.0000x reference)
import functools

import jax
import jax.numpy as jnp
from jax import lax
from jax.experimental import pallas as pl
from jax.experimental.pallas import tpu as pltpu
from jax.experimental.pallas import tpu_sc as plsc

NBINS = 128 * 128 * 128
HALF = NBINS // 2
DUMP = HALF
ACC_SIZE = HALF + 256

NPTS = 4_000_000
NTILES = 16
PTS_PER_TILE = NPTS // NTILES
BATCH_PTS = 10000
GROUPS = BATCH_PTS // 16
NBATCH = PTS_PER_TILE // BATCH_PTS
CS = 512
IDX_CAP = BATCH_PTS + CS

OUT_PER_TILE = HALF // NTILES
CHUNK = 4096
NCHUNK = OUT_PER_TILE // CHUNK


def _sc_body(x_hbm, z0_hbm, z1_hbm, z2_hbm, counts_hbm, r_hbm,
             acc_sp, zb0, zb1, zb2, idx_v, ones_v, zeros_v, cnt_v, x_v, r_v):
    c = lax.axis_index("c")
    s = lax.axis_index("s")
    zero16 = jnp.zeros((16,), jnp.float32)
    one16 = jnp.ones((16,), jnp.float32)
    dump16 = jnp.full((16,), DUMP, jnp.int32)

    def init_zeros(i, _):
        zeros_v[pl.ds(i * 16, 16)] = zero16
        return 0
    lax.fori_loop(0, CHUNK // 16, init_zeros, 0)

    def init_ones(i, _):
        ones_v[pl.ds(i * 16, 16)] = one16
        return 0
    lax.fori_loop(0, CS // 16, init_ones, 0)

    def zero_main(i, _):
        pltpu.sync_copy(zeros_v, acc_sp.at[pl.ds(s * OUT_PER_TILE + i * CHUNK,
                                                 CHUNK)])
        return 0
    lax.fori_loop(0, NCHUNK, zero_main, 0)

    @pl.when(s == 0)
    def _():
        pltpu.sync_copy(zeros_v.at[pl.ds(0, 256)], acc_sp.at[pl.ds(HALF, 256)])

    plsc.subcore_barrier()

    half_lo = c * HALF

    def batch_body(b, _):
        pbase = s * PTS_PER_TILE + b * BATCH_PTS

        def group_body(g, pos):
            off = pl.ds(g * 16, 16)
            b0 = zb0[off].astype(jnp.int32)
            b1 = zb1[off].astype(jnp.int32)
            b2 = zb2[off].astype(jnp.int32)
            flat = (b0 << 14) + (b1 << 7) + b2
            local = flat - half_lo
            mine = local.astype(jnp.uint32) < jnp.uint32(HALF)
            plsc.store_compressed(idx_v.at[pl.ds(pos, 16)], local, mask=mine)
            return pos + plsc.all_reduce_population_count(mine)[0]
        pos = lax.fori_loop(0, GROUPS, group_body, 0, unroll=8)

        for k in range(CS // 16):
            idx_v[pl.ds(pos + k * 16, 16)] = dump16

        nchunks = (pos + CS - 1) // CS

        def scat_cond(r):
            return r < nchunks

        def scat_body(r):
            pltpu.sync_copy(ones_v,
                            acc_sp.at[idx_v.at[pl.ds(r * CS, CS)]], add=True)
            return r + 1
        lax.while_loop(scat_cond, scat_body, 0)
        return 0
    lax.fori_loop(0, NBATCH, batch_body, 0)

    plsc.subcore_barrier()

    def chunk_body(i, _):
        sbase = s * OUT_PER_TILE + i * CHUNK
        gbase = c * HALF + sbase
        pltpu.sync_copy(acc_sp.at[pl.ds(sbase, CHUNK)], cnt_v)
        pltpu.sync_copy(x_hbm.at[pl.ds(gbase, CHUNK)], x_v)

        def mask_body(k, _):
            cc = cnt_v[pl.ds(k * 16, 16)]
            xx = x_v[pl.ds(k * 16, 16)]
            r_v[pl.ds(k * 16, 16)] = jnp.where(cc > 0.0, xx, zero16)
            return 0
        lax.fori_loop(0, CHUNK // 16, mask_body, 0, unroll=8)

        pltpu.sync_copy(cnt_v, counts_hbm.at[pl.ds(gbase, CHUNK)])
        pltpu.sync_copy(r_v, r_hbm.at[pl.ds(gbase, CHUNK)])
        return 0
    lax.fori_loop(0, NCHUNK, chunk_body, 0)


@jax.jit
def _run(x_flat, z0, z1, z2):
    mesh = plsc.VectorSubcoreMesh(core_axis_name="c", subcore_axis_name="s")
    kfn = pl.kernel(
        _sc_body,
        out_type=[jax.ShapeDtypeStruct((NBINS,), jnp.float32),
                  jax.ShapeDtypeStruct((NBINS,), jnp.float32)],
        mesh=mesh,
        compiler_params=pltpu.CompilerParams(needs_layout_passes=False),
        scratch_types=[
            pltpu.VMEM_SHARED((ACC_SIZE,), jnp.float32),
            pltpu.VMEM((BATCH_PTS + 240,), jnp.float32),
            pltpu.VMEM((BATCH_PTS + 240,), jnp.float32),
            pltpu.VMEM((BATCH_PTS + 240,), jnp.float32),
            pltpu.VMEM((IDX_CAP,), jnp.int32),
            pltpu.VMEM((CS,), jnp.float32),
            pltpu.VMEM((CHUNK,), jnp.float32),
            pltpu.VMEM((CHUNK,), jnp.float32),
            pltpu.VMEM((CHUNK,), jnp.float32),
            pltpu.VMEM((CHUNK,), jnp.float32),
        ],
    )
    return kfn(x_flat, z0, z1, z2)


def kernel(x, z):
    z0 = z[:, 0] * 128.0
    z1 = z[:, 1] * 128.0
    z2 = z[:, 2] * 128.0
    counts, r = _run(x.reshape(-1), z0, z1, z2)
    return counts.reshape(x.shape), r.reshape(x.shape)


# speedup vs baseline: 3.3770x; 3.3770x over previous
import functools

import jax
import jax.numpy as jnp
from jax import lax
from jax.experimental import pallas as pl
from jax.experimental.pallas import tpu as pltpu
from jax.experimental.pallas import tpu_sc as plsc

NBINS = 128 * 128 * 128
HALF = NBINS // 2
DUMP = HALF
ACC_SIZE = HALF + 256

NPTS = 4_000_000
NTILES = 16
PTS_PER_TILE = NPTS // NTILES
BATCH_PTS = 2000
GROUPS = BATCH_PTS // 16
QGROUPS = GROUPS // 4
NBATCH = PTS_PER_TILE // BATCH_PTS
NPAIR = NBATCH // 2
CS = 256
QCAP = BATCH_PTS // 4 + 16 + CS

OUT_PER_TILE = HALF // NTILES
CHUNK = 2048
NCHUNK = OUT_PER_TILE // CHUNK


def _sc_body(x_hbm, z0_hbm, z1_hbm, z2_hbm, counts_hbm, r_hbm,
             acc_sp, za0, za1, za2, zb0, zb1, zb2,
             q0, q1, q2, q3, ones_v, zeros_v, cnt_v, x_v, r_v, zsem):
    c = lax.axis_index("c")
    s = lax.axis_index("s")
    zero16 = jnp.zeros((16,), jnp.float32)
    one16 = jnp.ones((16,), jnp.float32)
    dump16 = jnp.full((16,), DUMP, jnp.int32)
    qrefs = (q0, q1, q2, q3)

    def init_zeros(i, _):
        zeros_v[pl.ds(i * 16, 16)] = zero16
        return 0
    lax.fori_loop(0, CHUNK // 16, init_zeros, 0)

    def init_ones(i, _):
        ones_v[pl.ds(i * 16, 16)] = one16
        return 0
    lax.fori_loop(0, CS // 16, init_ones, 0)

    def zero_main(i, _):
        pltpu.sync_copy(zeros_v, acc_sp.at[pl.ds(s * OUT_PER_TILE + i * CHUNK,
                                                 CHUNK)])
        return 0
    lax.fori_loop(0, NCHUNK, zero_main, 0)

    @pl.when(s == 0)
    def _():
        pltpu.sync_copy(zeros_v.at[pl.ds(0, 256)], acc_sp.at[pl.ds(HALF, 256)])

    plsc.subcore_barrier()

    half_lo = c * HALF

    def z_issue(b, d0, d1, d2):
        pbase = s * PTS_PER_TILE + b * BATCH_PTS
        pltpu.async_copy(z0_hbm.at[pl.ds(pbase, BATCH_PTS)],
                         d0.at[pl.ds(0, BATCH_PTS)], zsem)
        pltpu.async_copy(z1_hbm.at[pl.ds(pbase, BATCH_PTS)],
                         d1.at[pl.ds(0, BATCH_PTS)], zsem)
        pltpu.async_copy(z2_hbm.at[pl.ds(pbase, BATCH_PTS)],
                         d2.at[pl.ds(0, BATCH_PTS)], zsem)

    def z_wait(d0, d1, d2):
        pltpu.make_async_copy(z0_hbm.at[pl.ds(0, BATCH_PTS)],
                              d0.at[pl.ds(0, BATCH_PTS)], zsem).wait()
        pltpu.make_async_copy(z1_hbm.at[pl.ds(0, BATCH_PTS)],
                              d1.at[pl.ds(0, BATCH_PTS)], zsem).wait()
        pltpu.make_async_copy(z2_hbm.at[pl.ds(0, BATCH_PTS)],
                              d2.at[pl.ds(0, BATCH_PTS)], zsem).wait()

    def one_group(g, d0, d1, d2, qref, pos):
        off = pl.ds(g * 16, 16)
        b0 = d0[off].astype(jnp.int32)
        b1 = d1[off].astype(jnp.int32)
        b2 = d2[off].astype(jnp.int32)
        flat = (b0 << 14) + (b1 << 7) + b2
        local = flat - half_lo
        mine = local.astype(jnp.uint32) < jnp.uint32(HALF)
        plsc.store_compressed(qref.at[pl.ds(pos, 16)], local, mask=mine)
        return pos + plsc.all_reduce_population_count(mine)[0]

    def compute_scatter(d0, d1, d2):
        def group4(i, ps):
            new_ps = []
            for q in range(4):
                new_ps.append(one_group(i * 4 + q, d0, d1, d2,
                                        qrefs[q], ps[q]))
            return tuple(new_ps)
        ps = lax.fori_loop(0, QGROUPS, group4, (0, 0, 0, 0))
        p0 = one_group(GROUPS - 1, d0, d1, d2, q0, ps[0])
        ps = (p0, ps[1], ps[2], ps[3])

        for q in range(4):
            pos = ps[q]
            qref = qrefs[q]
            for k in range(CS // 16):
                qref[pl.ds(pos + k * 16, 16)] = dump16

            nchunks = (pos + CS - 1) // CS

            def scat_body(r, qref=qref):
                pltpu.sync_copy(ones_v,
                                acc_sp.at[qref.at[pl.ds(r * CS, CS)]],
                                add=True)
                return r + 1
            lax.while_loop(lambda r, n=nchunks: r < n, scat_body, 0)

    z_issue(0, za0, za1, za2)

    def pair_body(p, _):
        z_wait(za0, za1, za2)
        z_issue(2 * p + 1, zb0, zb1, zb2)
        compute_scatter(za0, za1, za2)
        z_wait(zb0, zb1, zb2)
        z_issue(2 * p + 2, za0, za1, za2)
        compute_scatter(zb0, zb1, zb2)
        return 0
    lax.fori_loop(0, NPAIR, pair_body, 0)

    z_wait(za0, za1, za2)
    compute_scatter(za0, za1, za2)

    plsc.subcore_barrier()

    def chunk_body(i, _):
        sbase = s * OUT_PER_TILE + i * CHUNK
        gbase = c * HALF + sbase
        pltpu.sync_copy(acc_sp.at[pl.ds(sbase, CHUNK)], cnt_v)
        pltpu.sync_copy(x_hbm.at[pl.ds(gbase, CHUNK)], x_v)

        def mask_body(k, _):
            cc = cnt_v[pl.ds(k * 16, 16)]
            xx = x_v[pl.ds(k * 16, 16)]
            r_v[pl.ds(k * 16, 16)] = jnp.where(cc > 0.0, xx, zero16)
            return 0
        lax.fori_loop(0, CHUNK // 16, mask_body, 0, unroll=8)

        pltpu.sync_copy(cnt_v, counts_hbm.at[pl.ds(gbase, CHUNK)])
        pltpu.sync_copy(r_v, r_hbm.at[pl.ds(gbase, CHUNK)])
        return 0
    lax.fori_loop(0, NCHUNK, chunk_body, 0)


@jax.jit
def _run(x_flat, z0, z1, z2):
    mesh = plsc.VectorSubcoreMesh(core_axis_name="c", subcore_axis_name="s")
    kfn = pl.kernel(
        _sc_body,
        out_type=[jax.ShapeDtypeStruct((NBINS,), jnp.float32),
                  jax.ShapeDtypeStruct((NBINS,), jnp.float32)],
        mesh=mesh,
        compiler_params=pltpu.CompilerParams(needs_layout_passes=False),
        scratch_types=[
            pltpu.VMEM_SHARED((ACC_SIZE,), jnp.float32),
            pltpu.VMEM((BATCH_PTS + 240,), jnp.float32),
            pltpu.VMEM((BATCH_PTS + 240,), jnp.float32),
            pltpu.VMEM((BATCH_PTS + 240,), jnp.float32),
            pltpu.VMEM((BATCH_PTS + 240,), jnp.float32),
            pltpu.VMEM((BATCH_PTS + 240,), jnp.float32),
            pltpu.VMEM((BATCH_PTS + 240,), jnp.float32),
            pltpu.VMEM((QCAP,), jnp.int32),
            pltpu.VMEM((QCAP,), jnp.int32),
            pltpu.VMEM((QCAP,), jnp.int32),
            pltpu.VMEM((QCAP,), jnp.int32),
            pltpu.VMEM((CS,), jnp.float32),
            pltpu.VMEM((CHUNK,), jnp.float32),
            pltpu.VMEM((CHUNK,), jnp.float32),
            pltpu.VMEM((CHUNK,), jnp.float32),
            pltpu.VMEM((CHUNK,), jnp.float32),
            pltpu.SemaphoreType.DMA,
        ],
    )
    return kfn(x_flat, z0, z1, z2)


def kernel(x, z):
    z0 = z[:, 0] * 128.0
    z1 = z[:, 1] * 128.0
    z2 = z[:, 2] * 128.0
    counts, r = _run(x.reshape(-1), z0, z1, z2)
    return counts.reshape(x.shape), r.reshape(x.shape)


# speedup vs baseline: 4.2505x; 1.2587x over previous
import functools

import jax
import jax.numpy as jnp
from jax import lax
from jax.experimental import pallas as pl
from jax.experimental.pallas import tpu as pltpu
from jax.experimental.pallas import tpu_sc as plsc

NBINS = 128 * 128 * 128
HALF = NBINS // 2
DUMP = HALF
ACC_SIZE = HALF + 256

NPTS = 4_000_000
NTILES = 16
PTS_PER_TILE = NPTS // NTILES
BATCH_PTS = 10000
GROUPS = BATCH_PTS // 16
QGROUPS = GROUPS // 4
NBATCH = PTS_PER_TILE // BATCH_PTS
NPAIR = NBATCH // 2
CS = 512
QCAP = BATCH_PTS // 4 + 16 + CS

OUT_PER_TILE = HALF // NTILES
CHUNK = 2048
NCHUNK = OUT_PER_TILE // CHUNK


def _sc_body(x_hbm, z0_hbm, z1_hbm, z2_hbm, counts_hbm, r_hbm,
             acc_sp, za0, za1, za2,
             q0, q1, q2, q3, ones_v, zeros_v, cnt_v, x_v, r_v, zsem):
    c = lax.axis_index("c")
    s = lax.axis_index("s")
    zero16 = jnp.zeros((16,), jnp.float32)
    one16 = jnp.ones((16,), jnp.float32)
    dump16 = jnp.full((16,), DUMP, jnp.int32)
    qrefs = (q0, q1, q2, q3)

    def init_zeros(i, _):
        zeros_v[pl.ds(i * 16, 16)] = zero16
        return 0
    lax.fori_loop(0, CHUNK // 16, init_zeros, 0)

    def init_ones(i, _):
        ones_v[pl.ds(i * 16, 16)] = one16
        return 0
    lax.fori_loop(0, CS // 16, init_ones, 0)

    def zero_main(i, _):
        pltpu.sync_copy(zeros_v, acc_sp.at[pl.ds(s * OUT_PER_TILE + i * CHUNK,
                                                 CHUNK)])
        return 0
    lax.fori_loop(0, NCHUNK, zero_main, 0)

    @pl.when(s == 0)
    def _():
        pltpu.sync_copy(zeros_v.at[pl.ds(0, 256)], acc_sp.at[pl.ds(HALF, 256)])

    plsc.subcore_barrier()

    half_lo = c * HALF

    def z_issue(b, d0, d1, d2):
        pbase = s * PTS_PER_TILE + b * BATCH_PTS
        pltpu.async_copy(z0_hbm.at[pl.ds(pbase, BATCH_PTS)],
                         d0.at[pl.ds(0, BATCH_PTS)], zsem)
        pltpu.async_copy(z1_hbm.at[pl.ds(pbase, BATCH_PTS)],
                         d1.at[pl.ds(0, BATCH_PTS)], zsem)
        pltpu.async_copy(z2_hbm.at[pl.ds(pbase, BATCH_PTS)],
                         d2.at[pl.ds(0, BATCH_PTS)], zsem)

    def z_wait(d0, d1, d2):
        pltpu.make_async_copy(z0_hbm.at[pl.ds(0, BATCH_PTS)],
                              d0.at[pl.ds(0, BATCH_PTS)], zsem).wait()
        pltpu.make_async_copy(z1_hbm.at[pl.ds(0, BATCH_PTS)],
                              d1.at[pl.ds(0, BATCH_PTS)], zsem).wait()
        pltpu.make_async_copy(z2_hbm.at[pl.ds(0, BATCH_PTS)],
                              d2.at[pl.ds(0, BATCH_PTS)], zsem).wait()

    def one_group(g, d0, d1, d2, qref, pos):
        off = pl.ds(g * 16, 16)
        b0 = d0[off].astype(jnp.int32)
        b1 = d1[off].astype(jnp.int32)
        b2 = d2[off].astype(jnp.int32)
        flat = (b0 << 14) + (b1 << 7) + b2
        local = flat - half_lo
        mine = local.astype(jnp.uint32) < jnp.uint32(HALF)
        plsc.store_compressed(qref.at[pl.ds(pos, 16)], local, mask=mine)
        return pos + plsc.all_reduce_population_count(mine)[0]

    def compute_scatter(d0, d1, d2):
        def group4(i, ps):
            new_ps = []
            for q in range(4):
                new_ps.append(one_group(i * 4 + q, d0, d1, d2,
                                        qrefs[q], ps[q]))
            return tuple(new_ps)
        ps = lax.fori_loop(0, QGROUPS, group4, (0, 0, 0, 0))
        p0 = one_group(GROUPS - 1, d0, d1, d2, q0, ps[0])
        ps = (p0, ps[1], ps[2], ps[3])

        for q in range(4):
            pos = ps[q]
            qref = qrefs[q]
            for k in range(CS // 16):
                qref[pl.ds(pos + k * 16, 16)] = dump16

            nchunks = (pos + CS - 1) // CS

            def scat_body(r, qref=qref):
                pltpu.sync_copy(ones_v,
                                acc_sp.at[qref.at[pl.ds(r * CS, CS)]],
                                add=True)
                return r + 1
            lax.while_loop(lambda r, n=nchunks: r < n, scat_body, 0)

    def batch_body(b, _):
        z_issue(b, za0, za1, za2)
        z_wait(za0, za1, za2)
        compute_scatter(za0, za1, za2)
        return 0
    lax.fori_loop(0, NBATCH, batch_body, 0)

    plsc.subcore_barrier()

    def chunk_body(i, _):
        sbase = s * OUT_PER_TILE + i * CHUNK
        gbase = c * HALF + sbase
        pltpu.sync_copy(acc_sp.at[pl.ds(sbase, CHUNK)], cnt_v)
        pltpu.sync_copy(x_hbm.at[pl.ds(gbase, CHUNK)], x_v)

        def mask_body(k, _):
            cc = cnt_v[pl.ds(k * 16, 16)]
            xx = x_v[pl.ds(k * 16, 16)]
            r_v[pl.ds(k * 16, 16)] = jnp.where(cc > 0.0, xx, zero16)
            return 0
        lax.fori_loop(0, CHUNK // 16, mask_body, 0, unroll=8)

        pltpu.sync_copy(cnt_v, counts_hbm.at[pl.ds(gbase, CHUNK)])
        pltpu.sync_copy(r_v, r_hbm.at[pl.ds(gbase, CHUNK)])
        return 0
    lax.fori_loop(0, NCHUNK, chunk_body, 0)


@jax.jit
def _run(x_flat, z0, z1, z2):
    mesh = plsc.VectorSubcoreMesh(core_axis_name="c", subcore_axis_name="s")
    kfn = pl.kernel(
        _sc_body,
        out_type=[jax.ShapeDtypeStruct((NBINS,), jnp.float32),
                  jax.ShapeDtypeStruct((NBINS,), jnp.float32)],
        mesh=mesh,
        compiler_params=pltpu.CompilerParams(needs_layout_passes=False),
        scratch_types=[
            pltpu.VMEM_SHARED((ACC_SIZE,), jnp.float32),
            pltpu.VMEM((BATCH_PTS + 240,), jnp.float32),
            pltpu.VMEM((BATCH_PTS + 240,), jnp.float32),
            pltpu.VMEM((BATCH_PTS + 240,), jnp.float32),
            pltpu.VMEM((QCAP,), jnp.int32),
            pltpu.VMEM((QCAP,), jnp.int32),
            pltpu.VMEM((QCAP,), jnp.int32),
            pltpu.VMEM((QCAP,), jnp.int32),
            pltpu.VMEM((CS,), jnp.float32),
            pltpu.VMEM((CHUNK,), jnp.float32),
            pltpu.VMEM((CHUNK,), jnp.float32),
            pltpu.VMEM((CHUNK,), jnp.float32),
            pltpu.VMEM((CHUNK,), jnp.float32),
            pltpu.SemaphoreType.DMA,
        ],
    )
    return kfn(x_flat, z0, z1, z2)


def kernel(x, z):
    z0 = z[:, 0] * 128.0
    z1 = z[:, 1] * 128.0
    z2 = z[:, 2] * 128.0
    counts, r = _run(x.reshape(-1), z0, z1, z2)
    return counts.reshape(x.shape), r.reshape(x.shape)


# speedup vs baseline: 6.6368x; 1.5614x over previous
import functools

import jax
import jax.numpy as jnp
from jax import lax
from jax.experimental import pallas as pl
from jax.experimental.pallas import tpu as pltpu
from jax.experimental.pallas import tpu_sc as plsc

NBINS = 128 * 128 * 128
HALF = NBINS // 2
DUMP = HALF
ACC_SIZE = HALF + 256

NPTS = 4_000_000
NTILES = 16
PTS_PER_TILE = NPTS // NTILES
BATCH_PTS = 10000
GROUPS = BATCH_PTS // 16
NBATCH = PTS_PER_TILE // BATCH_PTS
CS = 512
IDX_CAP = BATCH_PTS + CS
SUB_PTS = 2000
NRING = BATCH_PTS // SUB_PTS
SUB_GROUPS = SUB_PTS // 16
NSUB = PTS_PER_TILE // SUB_PTS

OUT_PER_TILE = HALF // NTILES
CHUNK = 4096
NCHUNK = OUT_PER_TILE // CHUNK


def _sc_body(x_hbm, z0_hbm, z1_hbm, z2_hbm, counts_hbm, r_hbm,
             acc_sp, zb0, zb1, zb2, idx_v, ones_v, zeros_v, cnt_v, x_v, r_v,
             zsem, zsem2):
    c = lax.axis_index("c")
    s = lax.axis_index("s")
    zero16 = jnp.zeros((16,), jnp.float32)
    one16 = jnp.ones((16,), jnp.float32)
    dump16 = jnp.full((16,), DUMP, jnp.int32)

    def init_zeros(i, _):
        zeros_v[pl.ds(i * 16, 16)] = zero16
        return 0
    lax.fori_loop(0, CHUNK // 16, init_zeros, 0)

    def init_ones(i, _):
        ones_v[pl.ds(i * 16, 16)] = one16
        return 0
    lax.fori_loop(0, CS // 16, init_ones, 0)

    def zero_main(i, _):
        pltpu.sync_copy(zeros_v, acc_sp.at[pl.ds(s * OUT_PER_TILE + i * CHUNK,
                                                 CHUNK)])
        return 0
    lax.fori_loop(0, NCHUNK, zero_main, 0)

    @pl.when(s == 0)
    def _():
        pltpu.sync_copy(zeros_v.at[pl.ds(0, 256)], acc_sp.at[pl.ds(HALF, 256)])

    plsc.subcore_barrier()

    half_lo = c * HALF

    def z_issue(k, sem):
        pbase = s * PTS_PER_TILE + k * SUB_PTS
        roff = (k % NRING) * SUB_PTS
        pltpu.async_copy(z0_hbm.at[pl.ds(pbase, SUB_PTS)],
                         zb0.at[pl.ds(roff, SUB_PTS)], sem)
        pltpu.async_copy(z1_hbm.at[pl.ds(pbase, SUB_PTS)],
                         zb1.at[pl.ds(roff, SUB_PTS)], sem)
        pltpu.async_copy(z2_hbm.at[pl.ds(pbase, SUB_PTS)],
                         zb2.at[pl.ds(roff, SUB_PTS)], sem)

    def z_wait(sem):
        pltpu.make_async_copy(z0_hbm.at[pl.ds(0, 3 * SUB_PTS)],
                              zb0.at[pl.ds(0, 3 * SUB_PTS)], sem).wait()

    z_issue(0, zsem)

    def do_sub(k, sem_w, sem_i, state):
        pos, flushed = state
        z_wait(sem_w)

        @pl.when(k + 1 < NSUB)
        def _():
            z_issue(k + 1, sem_i)

        roff = (k % NRING) * SUB_PTS

        def group_body(g, pos):
            off = pl.ds(roff + g * 16, 16)
            b0 = zb0[off].astype(jnp.int32)
            b1 = zb1[off].astype(jnp.int32)
            b2 = zb2[off].astype(jnp.int32)
            flat = (b0 << 14) + (b1 << 7) + b2
            local = flat - half_lo
            mine = local.astype(jnp.uint32) < jnp.uint32(HALF)
            plsc.store_compressed(idx_v.at[pl.ds(pos, 16)], local, mask=mine)
            return pos + plsc.all_reduce_population_count(mine)[0]
        pos = lax.fori_loop(0, SUB_GROUPS, group_body, pos)

        def flush_cond(st):
            return (st[0] + 1) * CS <= pos

        def flush_body(st):
            r, _ = st
            pltpu.sync_copy(ones_v,
                            acc_sp.at[idx_v.at[pl.ds(r * CS, CS)]], add=True)
            return (r + 1, 0)
        flushed, _ = lax.while_loop(flush_cond, flush_body, (flushed, 0))

        def wrap(st):
            pos, flushed = st
            tail = pos - flushed * CS

            def mv(i, _):
                v = idx_v[pl.ds(flushed * CS + i * 16, 16)]
                idx_v[pl.ds(i * 16, 16)] = v
                return 0
            lax.fori_loop(0, (tail + 15) // 16, mv, 0)
            return (tail, 0)

        state = lax.cond(pos + SUB_PTS + 16 > IDX_CAP,
                         wrap, lambda st: st, (pos, flushed))
        return state

    def pair_body(p, state):
        state = do_sub(2 * p, zsem, zsem2, state)
        state = do_sub(2 * p + 1, zsem2, zsem, state)
        return state
    state = lax.fori_loop(0, NSUB // 2, pair_body, (0, 0))
    pos, flushed = do_sub(NSUB - 1, zsem, zsem2, state)

    base = flushed * CS
    tail = pos - base
    for k in range(CS // 16):
        idx_v[pl.ds(pos + k * 16, 16)] = dump16

    def fin_body(r):
        pltpu.sync_copy(ones_v,
                        acc_sp.at[idx_v.at[pl.ds(base + r * CS, CS)]],
                        add=True)
        return r + 1
    nfin = (tail + CS - 1) // CS
    lax.while_loop(lambda r: r < nfin, fin_body, 0)

    plsc.subcore_barrier()

    def chunk_body(i, _):
        sbase = s * OUT_PER_TILE + i * CHUNK
        gbase = c * HALF + sbase
        pltpu.sync_copy(acc_sp.at[pl.ds(sbase, CHUNK)], cnt_v)
        pltpu.sync_copy(x_hbm.at[pl.ds(gbase, CHUNK)], x_v)

        def mask_body(k, _):
            cc = cnt_v[pl.ds(k * 16, 16)]
            xx = x_v[pl.ds(k * 16, 16)]
            r_v[pl.ds(k * 16, 16)] = jnp.where(cc > 0.0, xx, zero16)
            return 0
        lax.fori_loop(0, CHUNK // 16, mask_body, 0, unroll=8)

        pltpu.sync_copy(cnt_v, counts_hbm.at[pl.ds(gbase, CHUNK)])
        pltpu.sync_copy(r_v, r_hbm.at[pl.ds(gbase, CHUNK)])
        return 0
    lax.fori_loop(0, NCHUNK, chunk_body, 0)


@jax.jit
def _run(x_flat, z0, z1, z2):
    mesh = plsc.VectorSubcoreMesh(core_axis_name="c", subcore_axis_name="s")
    kfn = pl.kernel(
        _sc_body,
        out_type=[jax.ShapeDtypeStruct((NBINS,), jnp.float32),
                  jax.ShapeDtypeStruct((NBINS,), jnp.float32)],
        mesh=mesh,
        compiler_params=pltpu.CompilerParams(needs_layout_passes=False),
        scratch_types=[
            pltpu.VMEM_SHARED((ACC_SIZE,), jnp.float32),
            pltpu.VMEM((BATCH_PTS + 240,), jnp.float32),
            pltpu.VMEM((BATCH_PTS + 240,), jnp.float32),
            pltpu.VMEM((BATCH_PTS + 240,), jnp.float32),
            pltpu.VMEM((IDX_CAP,), jnp.int32),
            pltpu.VMEM((CS,), jnp.float32),
            pltpu.VMEM((CHUNK,), jnp.float32),
            pltpu.VMEM((CHUNK,), jnp.float32),
            pltpu.VMEM((CHUNK,), jnp.float32),
            pltpu.VMEM((CHUNK,), jnp.float32),
            pltpu.SemaphoreType.DMA,
            pltpu.SemaphoreType.DMA,
        ],
    )
    return kfn(x_flat, z0, z1, z2)


def kernel(x, z):
    z0 = z[:, 0] * 128.0
    z1 = z[:, 1] * 128.0
    z2 = z[:, 2] * 128.0
    counts, r = _run(x.reshape(-1), z0, z1, z2)
    return counts.reshape(x.shape), r.reshape(x.shape)
